# baseline (speedup 1.0000x reference)
import functools

import numpy as np
import jax
import jax.numpy as jnp
from jax import lax
from jax.experimental import pallas as pl
from jax.experimental.pallas import tpu as pltpu
from jax.experimental.pallas import tpu_sc as plsc

_MIN_DISABLED = 2
_MAX_DISABLED = 8
_NEIGHBOR_PROB = 0.5


def _disabled_tofs(tof_count):
    rng = np.random.default_rng(0)
    disabled_count = int(rng.integers(_MIN_DISABLED, _MAX_DISABLED + 1))
    initial = int(rng.integers(0, tof_count))
    disabled = [initial]
    tof_list = [int(t) for t in rng.permutation(tof_count) if int(t) != initial]
    for _ in range(disabled_count - 1):
        rv = float(rng.random())
        perm = rng.permutation(len(disabled))
        permuted = [disabled[int(j)] for j in perm]
        if rv < _NEIGHBOR_PROB:
            if rv < _NEIGHBOR_PROB / 2:
                for cur in permuted:
                    new_neighbor = (cur + 1) % tof_count
                    if new_neighbor not in disabled:
                        disabled.append(new_neighbor)
                        tof_list = [t for t in tof_list if t != new_neighbor]
                        break
            else:
                opposite_found = False
                for cur in permuted:
                    new_opposite = (cur + tof_count // 2) % tof_count
                    if new_opposite not in disabled:
                        disabled.append(new_opposite)
                        tof_list = [t for t in tof_list if t != new_opposite]
                        opposite_found = True
                        break
                if not opposite_found:
                    new_element = tof_list[0]
                    tof_list = [t for t in tof_list if t != new_element]
                    disabled.append(new_element)
        else:
            new_element = tof_list[0]
            tof_list = [t for t in tof_list if t != new_element]
            disabled.append(new_element)
    return sorted(set(int(d) for d in disabled))


_ROWS, _COLS = 65536, 512
_DISABLED = _disabled_tofs(_COLS)

_NC, _NS, _L = 2, 16, 16
_NW = _NC * _NS
_RPW = _ROWS // _NW
_R = 56
_NB = 4
_CHUNKS = []
_off = 0
while _off < _RPW:
    _CHUNKS.append((_off, min(_R, _RPW - _off)))
    _off += _R
_NCHUNK = len(_CHUNKS)


def _body(img_hbm, out_hbm, *refs):
    bufs = refs[:_NB]
    sins = refs[_NB:2 * _NB]
    souts = refs[2 * _NB:3 * _NB]

    wid = lax.axis_index("s") * _NC + lax.axis_index("c")
    base = wid * _RPW

    lane = lax.iota(jnp.int32, _L)
    groups = []
    for g0 in sorted({(c // _L) * _L for c in _DISABLED}):
        cond = None
        for c in _DISABLED:
            if c // _L == g0 // _L:
                eq = lane == (c - g0)
                cond = eq if cond is None else (cond | eq)
        groups.append((g0, cond))

    def _buf(i):
        rows = _CHUNKS[i][1]
        b = bufs[i % _NB]
        return b if rows == _R else b.at[pl.ds(0, rows)]

    def in_copy(i):
        off, rows = _CHUNKS[i]
        return pltpu.make_async_copy(
            img_hbm.at[pl.ds(base + off, rows)], _buf(i), sins[i % _NB])

    def out_copy(i):
        off, rows = _CHUNKS[i]
        return pltpu.make_async_copy(
            _buf(i), out_hbm.at[pl.ds(base + off, rows)], souts[i % _NB])

    def blend(i):
        buf = bufs[i % _NB]

        def zero_row(r, carry):
            for g0, cond in groups:
                v = buf[r, pl.ds(g0, _L)]
                buf[r, pl.ds(g0, _L)] = jnp.where(cond, 0.0, v)
            return carry
        lax.fori_loop(0, _CHUNKS[i][1], zero_row, 0)

    for j in range(min(_NB - 1, _NCHUNK)):
        in_copy(j).start()
    for i in range(_NCHUNK):
        in_copy(i).wait()
        if i + _NB - 1 < _NCHUNK:
            if i >= 1:
                out_copy(i - 1).wait()
            in_copy(i + _NB - 1).start()
        blend(i)
        out_copy(i).start()
    for j in range(max(0, _NCHUNK - _NB), _NCHUNK):
        out_copy(j).wait()


def kernel(img):
    mesh = plsc.VectorSubcoreMesh(
        core_axis_name="c", subcore_axis_name="s",
        num_cores=_NC, num_subcores=_NS,
    )
    run = pl.kernel(
        _body,
        out_type=jax.ShapeDtypeStruct((_ROWS, _COLS), jnp.float32),
        mesh=mesh,
        scratch_types=(
            [pltpu.VMEM((_R, _COLS), jnp.float32)] * _NB
            + [pltpu.SemaphoreType.DMA] * (2 * _NB)
        ),
    )
    return run(img)


# speedup vs baseline: 1.0125x; 1.0125x over previous
import functools

import numpy as np
import jax
import jax.numpy as jnp
from jax import lax
from jax.experimental import pallas as pl
from jax.experimental.pallas import tpu as pltpu
from jax.experimental.pallas import tpu_sc as plsc

_MIN_DISABLED = 2
_MAX_DISABLED = 8
_NEIGHBOR_PROB = 0.5


def _disabled_tofs(tof_count):
    rng = np.random.default_rng(0)
    disabled_count = int(rng.integers(_MIN_DISABLED, _MAX_DISABLED + 1))
    initial = int(rng.integers(0, tof_count))
    disabled = [initial]
    tof_list = [int(t) for t in rng.permutation(tof_count) if int(t) != initial]
    for _ in range(disabled_count - 1):
        rv = float(rng.random())
        perm = rng.permutation(len(disabled))
        permuted = [disabled[int(j)] for j in perm]
        if rv < _NEIGHBOR_PROB:
            if rv < _NEIGHBOR_PROB / 2:
                for cur in permuted:
                    new_neighbor = (cur + 1) % tof_count
                    if new_neighbor not in disabled:
                        disabled.append(new_neighbor)
                        tof_list = [t for t in tof_list if t != new_neighbor]
                        break
            else:
                opposite_found = False
                for cur in permuted:
                    new_opposite = (cur + tof_count // 2) % tof_count
                    if new_opposite not in disabled:
                        disabled.append(new_opposite)
                        tof_list = [t for t in tof_list if t != new_opposite]
                        opposite_found = True
                        break
                if not opposite_found:
                    new_element = tof_list[0]
                    tof_list = [t for t in tof_list if t != new_element]
                    disabled.append(new_element)
        else:
            new_element = tof_list[0]
            tof_list = [t for t in tof_list if t != new_element]
            disabled.append(new_element)
    return sorted(set(int(d) for d in disabled))


_ROWS, _COLS = 65536, 512
_DISABLED = _disabled_tofs(_COLS)

_NC, _NS, _L = 2, 16, 16
_NW = _NC * _NS
_RPW = _ROWS // _NW
_R = 80
_NB = 3
_CHUNKS = []
_off = 0
while _off < _RPW:
    _CHUNKS.append((_off, min(_R, _RPW - _off)))
    _off += _R
_NCHUNK = len(_CHUNKS)


def _body(img_hbm, out_hbm, *refs):
    bufs = refs[:_NB]
    sins = refs[_NB:2 * _NB]
    souts = refs[2 * _NB:3 * _NB]

    wid = lax.axis_index("s") * _NC + lax.axis_index("c")
    base = wid * _RPW

    lane = lax.iota(jnp.int32, _L)
    groups = []
    for g0 in sorted({(c // _L) * _L for c in _DISABLED}):
        cond = None
        for c in _DISABLED:
            if c // _L == g0 // _L:
                eq = lane == (c - g0)
                cond = eq if cond is None else (cond | eq)
        groups.append((g0, cond))

    def _buf(i):
        rows = _CHUNKS[i][1]
        b = bufs[i % _NB]
        return b if rows == _R else b.at[pl.ds(0, rows)]

    def in_copy(i):
        off, rows = _CHUNKS[i]
        return pltpu.make_async_copy(
            img_hbm.at[pl.ds(base + off, rows)], _buf(i), sins[i % _NB])

    def out_copy(i):
        off, rows = _CHUNKS[i]
        return pltpu.make_async_copy(
            _buf(i), out_hbm.at[pl.ds(base + off, rows)], souts[i % _NB])

    def blend(i):
        buf = bufs[i % _NB]

        def zero_row(r, carry):
            for g0, cond in groups:
                v = buf[r, pl.ds(g0, _L)]
                buf[r, pl.ds(g0, _L)] = jnp.where(cond, 0.0, v)
            return carry
        lax.fori_loop(0, _CHUNKS[i][1], zero_row, 0)

    for j in range(min(_NB - 1, _NCHUNK)):
        in_copy(j).start()
    for i in range(_NCHUNK):
        in_copy(i).wait()
        if i + _NB - 1 < _NCHUNK:
            if i >= 1:
                out_copy(i - 1).wait()
            in_copy(i + _NB - 1).start()
        blend(i)
        out_copy(i).start()
    for j in range(max(0, _NCHUNK - _NB), _NCHUNK):
        out_copy(j).wait()


def kernel(img):
    mesh = plsc.VectorSubcoreMesh(
        core_axis_name="c", subcore_axis_name="s",
        num_cores=_NC, num_subcores=_NS,
    )
    run = pl.kernel(
        _body,
        out_type=jax.ShapeDtypeStruct((_ROWS, _COLS), jnp.float32),
        mesh=mesh,
        scratch_types=(
            [pltpu.VMEM((_R, _COLS), jnp.float32)] * _NB
            + [pltpu.SemaphoreType.DMA] * (2 * _NB)
        ),
    )
    return run(img)
